# baseline (speedup 1.0000x reference)
import functools

import jax
import jax.numpy as jnp
from jax import lax
from jax.experimental import pallas as pl
from jax.experimental.pallas import tpu as pltpu
from jax.experimental.pallas import tpu_sc as plsc

_NUM_CORES = 2
_NUM_SUBCORES = 16
_NW = _NUM_CORES * _NUM_SUBCORES
_BBLK = 128
_RING = 6
_AHEAD = 3


def _make_gather(n_chunks: int, jblocks: int, d: int):
    mesh = plsc.VectorSubcoreMesh(core_axis_name="c", subcore_axis_name="s")
    n = n_chunks
    r, a = _RING, _AHEAD
    du_hi = d // 8
    t_total = n * _NW // jblocks

    @functools.partial(
        pl.kernel,
        out_type=jax.ShapeDtypeStruct((t_total, du_hi, jblocks, 8, _BBLK),
                                      jnp.float32),
        mesh=mesh,
        scratch_types=[
            pltpu.VMEM((n, _BBLK), jnp.int32),
            pltpu.VMEM((r, _BBLK, d), jnp.float32),
            pltpu.VMEM((r, du_hi, 8, _BBLK), jnp.float32),
        ] + [pltpu.SemaphoreType.DMA] * r,
        compiler_params=pltpu.CompilerParams(use_tc_tiling_on_sc=False,
                                             needs_layout_passes=False),
    )
    def gather(table_hbm, idx_hbm, out_hbm, idx_v, rows_v, rt_v, *sems):
        wid = lax.axis_index("s") * _NUM_CORES + lax.axis_index("c")
        c0 = wid * n
        pltpu.sync_copy(idx_hbm.at[pl.ds(c0, n)], idx_v)

        lanes = lax.iota(jnp.int32, 16)
        d_hi = [(lanes + 16 * q) // 8 for q in range(d // 16)]
        d_lo = [(lanes + 16 * q) % 8 for q in range(d // 16)]

        def start_gather(k, slot):
            pltpu.async_copy(table_hbm.at[idx_v.at[k]], rows_v.at[slot],
                             sems[slot])

        def wait_slot(slot):
            pltpu.make_async_copy(table_hbm.at[idx_v.at[0]], rows_v.at[slot],
                                  sems[slot]).wait()

        def transpose(slot):
            rows = rows_v.at[slot]
            rt = rt_v.at[slot]

            def b_body(bp, carry):
                col = jnp.full((16,), bp, jnp.int32)
                for q in range(d // 16):
                    vals = rows[bp, pl.ds(q * 16, 16)]
                    plsc.store_scatter(rt, [d_hi[q], d_lo[q], col], vals)
                return carry

            lax.fori_loop(0, _BBLK, b_body, 0)

        def start_out(k, slot):
            c = c0 + k
            tt = c // jblocks
            jj = c % jblocks
            pltpu.async_copy(rt_v.at[slot], out_hbm.at[tt, :, jj], sems[slot])

        def full_step(k, slot, slot_a):
            wait_slot(slot_a)
            start_gather(k + a, slot_a)
            wait_slot(slot)
            transpose(slot)
            start_out(k, slot)

        for k in range(a):
            start_gather(k, k % r)
        for k in range(r - a):
            start_gather(k + a, (k + a) % r)
            wait_slot(k % r)
            transpose(k % r)
            start_out(k, k % r)

        main_iters = (n - r) // r
        rem = (n - r) % r

        def main_body(g, carry):
            k0 = (r - a) + g * r
            for b in range(r):
                full_step(k0 + b, (r - a + b) % r, (r + b) % r)
            return carry
        lax.fori_loop(0, main_iters, main_body, 0)

        for i in range(rem):
            k = (r - a) + main_iters * r + i
            full_step(k, k % r, (k + a) % r)
        for k in range(n - a, n):
            wait_slot(k % r)
            transpose(k % r)
            start_out(k, k % r)
        for slot in range(r):
            wait_slot(slot)

    return gather


def kernel(token_ids, embed):
    b, t = token_ids.shape
    d = embed.shape[1]
    assert d % 8 == 0 and b % _BBLK == 0 and (t * (b // _BBLK)) % _NW == 0
    jblocks = b // _BBLK
    n_chunks = t * jblocks // _NW
    idx4 = jnp.transpose(token_ids).reshape(t * jblocks, _BBLK)
    idx4 = idx4.astype(jnp.int32)
    p = _make_gather(n_chunks, jblocks, d)(embed, idx4)
    return p.transpose(2, 4, 0, 1, 3).reshape(b, t, d)


# speedup vs baseline: 1.3695x; 1.3695x over previous
import functools

import jax
import jax.numpy as jnp
from jax import lax
from jax.experimental import pallas as pl
from jax.experimental.pallas import tpu as pltpu
from jax.experimental.pallas import tpu_sc as plsc

_NUM_CORES = 2
_NUM_SUBCORES = 16
_NW = _NUM_CORES * _NUM_SUBCORES
_BBLK = 128
_RING = 6
_AHEAD = 3


def _make_gather(n_chunks: int, jblocks: int, d: int):
    mesh = plsc.VectorSubcoreMesh(core_axis_name="c", subcore_axis_name="s")
    n = n_chunks
    r, a = _RING, _AHEAD
    du_hi = d // 8
    t_total = n * _NW // jblocks

    @functools.partial(
        pl.kernel,
        out_type=jax.ShapeDtypeStruct((t_total, du_hi, jblocks, 8, _BBLK),
                                      jnp.float32),
        mesh=mesh,
        scratch_types=[
            pltpu.VMEM((n, _BBLK), jnp.int32),
            pltpu.VMEM((r, _BBLK, d), jnp.float32),
            pltpu.VMEM((r, du_hi, 8, _BBLK + 1), jnp.float32),
        ] + [pltpu.SemaphoreType.DMA] * r,
        compiler_params=pltpu.CompilerParams(use_tc_tiling_on_sc=False,
                                             needs_layout_passes=False),
    )
    def gather(table_hbm, idx_hbm, out_hbm, idx_v, rows_v, rt_v, *sems):
        wid = lax.axis_index("s") * _NUM_CORES + lax.axis_index("c")
        c0 = wid * n
        pltpu.sync_copy(idx_hbm.at[pl.ds(c0, n)], idx_v)

        lanes = lax.iota(jnp.int32, 16)
        d_hi = [(lanes + 16 * q) // 8 for q in range(d // 16)]
        d_lo = [(lanes + 16 * q) % 8 for q in range(d // 16)]

        def start_gather(k, slot):
            pltpu.async_copy(table_hbm.at[idx_v.at[k]], rows_v.at[slot],
                             sems[slot])

        def wait_slot(slot):
            pltpu.make_async_copy(table_hbm.at[idx_v.at[0]], rows_v.at[slot],
                                  sems[slot]).wait()

        def transpose(slot):
            rows = rows_v.at[slot]
            rt = rt_v.at[slot]

            def b_body(bp, carry):
                col = jnp.full((16,), bp, jnp.int32)
                for q in range(d // 16):
                    vals = rows[bp, pl.ds(q * 16, 16)]
                    plsc.store_scatter(rt, [d_hi[q], d_lo[q], col], vals)
                return carry

            lax.fori_loop(0, _BBLK, b_body, 0)

        def start_out(k, slot):
            c = c0 + k
            tt = c // jblocks
            jj = c % jblocks
            pltpu.async_copy(rt_v.at[slot, :, :, pl.ds(0, _BBLK)],
                             out_hbm.at[tt, :, jj], sems[slot])

        def full_step(k, slot, slot_a):
            wait_slot(slot_a)
            start_gather(k + a, slot_a)
            wait_slot(slot)
            transpose(slot)
            start_out(k, slot)

        for k in range(a):
            start_gather(k, k % r)
        for k in range(r - a):
            start_gather(k + a, (k + a) % r)
            wait_slot(k % r)
            transpose(k % r)
            start_out(k, k % r)

        main_iters = (n - r) // r
        rem = (n - r) % r

        def main_body(g, carry):
            k0 = (r - a) + g * r
            for b in range(r):
                full_step(k0 + b, (r - a + b) % r, (r + b) % r)
            return carry
        lax.fori_loop(0, main_iters, main_body, 0)

        for i in range(rem):
            k = (r - a) + main_iters * r + i
            full_step(k, k % r, (k + a) % r)
        for k in range(n - a, n):
            wait_slot(k % r)
            transpose(k % r)
            start_out(k, k % r)
        for slot in range(r):
            wait_slot(slot)

    return gather


def kernel(token_ids, embed):
    b, t = token_ids.shape
    d = embed.shape[1]
    assert d % 8 == 0 and b % _BBLK == 0 and (t * (b // _BBLK)) % _NW == 0
    jblocks = b // _BBLK
    n_chunks = t * jblocks // _NW
    idx4 = jnp.transpose(token_ids).reshape(t * jblocks, _BBLK)
    idx4 = idx4.astype(jnp.int32)
    p = _make_gather(n_chunks, jblocks, d)(embed, idx4)
    return p.transpose(2, 4, 0, 1, 3).reshape(b, t, d)


# speedup vs baseline: 1.6335x; 1.1928x over previous
import functools

import jax
import jax.numpy as jnp
from jax import lax
from jax.experimental import pallas as pl
from jax.experimental.pallas import tpu as pltpu
from jax.experimental.pallas import tpu_sc as plsc

_NUM_CORES = 2
_NUM_SUBCORES = 16
_NW = _NUM_CORES * _NUM_SUBCORES
_BBLK = 128
_DO_TRANSPOSE = True
_RING = 6
_AHEAD = 3


def _make_gather(n_chunks: int, jblocks: int, d: int):
    mesh = plsc.VectorSubcoreMesh(core_axis_name="c", subcore_axis_name="s")
    n = n_chunks
    r, a = _RING, _AHEAD
    du_hi = d // 8
    t_total = n * _NW // jblocks

    @functools.partial(
        pl.kernel,
        out_type=jax.ShapeDtypeStruct((t_total, du_hi, jblocks, 8, _BBLK),
                                      jnp.float32),
        mesh=mesh,
        scratch_types=[
            pltpu.VMEM((n, _BBLK), jnp.int32),
            pltpu.VMEM((r, _BBLK, d), jnp.float32),
            pltpu.VMEM((r, du_hi, 8, _BBLK + 1), jnp.float32),
        ] + [pltpu.SemaphoreType.DMA] * r,
        compiler_params=pltpu.CompilerParams(use_tc_tiling_on_sc=False,
                                             needs_layout_passes=False),
    )
    def gather(table_hbm, idx_hbm, out_hbm, idx_v, rows_v, rt_v, *sems):
        wid = lax.axis_index("s") * _NUM_CORES + lax.axis_index("c")
        c0 = wid * n
        pltpu.sync_copy(idx_hbm.at[pl.ds(c0, n)], idx_v)

        lanes = lax.iota(jnp.int32, 16)
        d_hi = [(lanes + 16 * q) // 8 for q in range(d // 16)]
        d_lo = [(lanes + 16 * q) % 8 for q in range(d // 16)]

        def start_gather(k, slot):
            pltpu.async_copy(table_hbm.at[idx_v.at[k]], rows_v.at[slot],
                             sems[slot])

        def wait_slot(slot):
            pltpu.make_async_copy(table_hbm.at[idx_v.at[0]], rows_v.at[slot],
                                  sems[slot]).wait()

        def transpose(slot):
            rows = rows_v.at[slot]
            rt = rt_v.at[slot]

            @plsc.parallel_loop(0, _BBLK, unroll=4)
            def b_body(bp):
                col = jnp.full((16,), bp, jnp.int32)
                for q in range(d // 16):
                    vals = rows[bp, pl.ds(q * 16, 16)]
                    plsc.store_scatter(rt, [d_hi[q], d_lo[q], col], vals)

        def start_out(k, slot):
            c = c0 + k
            tt = c // jblocks
            jj = c % jblocks
            pltpu.async_copy(rt_v.at[slot, :, :, pl.ds(0, _BBLK)],
                             out_hbm.at[tt, :, jj], sems[slot])

        def full_step(k, slot, slot_a):
            wait_slot(slot_a)
            start_gather(k + a, slot_a)
            wait_slot(slot)
            if _DO_TRANSPOSE:
                transpose(slot)
            start_out(k, slot)

        for k in range(a):
            start_gather(k, k % r)
        for k in range(r - a):
            start_gather(k + a, (k + a) % r)
            wait_slot(k % r)
            if _DO_TRANSPOSE:
                transpose(k % r)
            start_out(k, k % r)

        main_iters = (n - r) // r
        rem = (n - r) % r

        def main_body(g, carry):
            k0 = (r - a) + g * r
            for b in range(r):
                full_step(k0 + b, (r - a + b) % r, (r + b) % r)
            return carry
        lax.fori_loop(0, main_iters, main_body, 0)

        for i in range(rem):
            k = (r - a) + main_iters * r + i
            full_step(k, k % r, (k + a) % r)
        for k in range(n - a, n):
            wait_slot(k % r)
            if _DO_TRANSPOSE:
                transpose(k % r)
            start_out(k, k % r)
        for slot in range(r):
            wait_slot(slot)

    return gather


def kernel(token_ids, embed):
    b, t = token_ids.shape
    d = embed.shape[1]
    assert d % 8 == 0 and b % _BBLK == 0 and (t * (b // _BBLK)) % _NW == 0
    jblocks = b // _BBLK
    n_chunks = t * jblocks // _NW
    idx4 = jnp.transpose(token_ids).reshape(t * jblocks, _BBLK)
    idx4 = idx4.astype(jnp.int32)
    p = _make_gather(n_chunks, jblocks, d)(embed, idx4)
    return p.transpose(2, 4, 0, 1, 3).reshape(b, t, d)


# speedup vs baseline: 1.6391x; 1.0034x over previous
import functools

import jax
import jax.numpy as jnp
from jax import lax
from jax.experimental import pallas as pl
from jax.experimental.pallas import tpu as pltpu
from jax.experimental.pallas import tpu_sc as plsc

_NUM_CORES = 2
_NUM_SUBCORES = 16
_NW = _NUM_CORES * _NUM_SUBCORES
_BBLK = 128
_RING = 6
_AHEAD = 4


def _make_gather(n_chunks: int, jblocks: int, d: int):
    mesh = plsc.VectorSubcoreMesh(core_axis_name="c", subcore_axis_name="s")
    n = n_chunks
    r, a = _RING, _AHEAD
    du_hi = d // 8
    t_total = n * _NW // jblocks

    @functools.partial(
        pl.kernel,
        out_type=jax.ShapeDtypeStruct((t_total, du_hi, jblocks, 8, _BBLK),
                                      jnp.float32),
        mesh=mesh,
        scratch_types=[
            pltpu.VMEM((n, _BBLK), jnp.int32),
            pltpu.VMEM((r, _BBLK, d), jnp.float32),
            pltpu.VMEM((r, du_hi, 8, _BBLK + 1), jnp.float32),
        ] + [pltpu.SemaphoreType.DMA] * r,
        compiler_params=pltpu.CompilerParams(use_tc_tiling_on_sc=False,
                                             needs_layout_passes=False),
    )
    def gather(table_hbm, idx_hbm, out_hbm, idx_v, rows_v, rt_v, *sems):
        wid = lax.axis_index("s") * _NUM_CORES + lax.axis_index("c")
        c0 = wid * n
        pltpu.sync_copy(idx_hbm.at[pl.ds(c0, n)], idx_v)

        lanes = lax.iota(jnp.int32, 16)
        d_hi = [(lanes + 16 * q) // 8 for q in range(d // 16)]
        d_lo = [(lanes + 16 * q) % 8 for q in range(d // 16)]

        def start_gather(k, slot):
            pltpu.async_copy(table_hbm.at[idx_v.at[k]], rows_v.at[slot],
                             sems[slot])

        def wait_slot(slot):
            pltpu.make_async_copy(table_hbm.at[idx_v.at[0]], rows_v.at[slot],
                                  sems[slot]).wait()

        def transpose(slot):
            rows = rows_v.at[slot]
            rt = rt_v.at[slot]

            @plsc.parallel_loop(0, _BBLK, unroll=4)
            def b_body(bp):
                col = jnp.full((16,), bp, jnp.int32)
                for q in range(d // 16):
                    vals = rows[bp, pl.ds(q * 16, 16)]
                    plsc.store_scatter(rt, [d_hi[q], d_lo[q], col], vals)

        def start_out(k, slot):
            c = c0 + k
            tt = c // jblocks
            jj = c % jblocks
            pltpu.async_copy(rt_v.at[slot, :, :, pl.ds(0, _BBLK)],
                             out_hbm.at[tt, :, jj], sems[slot])

        def full_step(k, slot, slot_a):
            wait_slot(slot_a)
            start_gather(k + a, slot_a)
            wait_slot(slot)
            transpose(slot)
            start_out(k, slot)

        for k in range(a):
            start_gather(k, k % r)
        for k in range(r - a):
            start_gather(k + a, (k + a) % r)
            wait_slot(k % r)
            transpose(k % r)
            start_out(k, k % r)

        main_iters = (n - r) // r
        rem = (n - r) % r

        def main_body(g, carry):
            k0 = (r - a) + g * r
            for b in range(r):
                full_step(k0 + b, (r - a + b) % r, (r + b) % r)
            return carry
        lax.fori_loop(0, main_iters, main_body, 0)

        for i in range(rem):
            k = (r - a) + main_iters * r + i
            full_step(k, k % r, (k + a) % r)
        for k in range(n - a, n):
            wait_slot(k % r)
            transpose(k % r)
            start_out(k, k % r)
        for slot in range(r):
            wait_slot(slot)

    return gather


def kernel(token_ids, embed):
    b, t = token_ids.shape
    d = embed.shape[1]
    assert d % 8 == 0 and b % _BBLK == 0 and (t * (b // _BBLK)) % _NW == 0
    jblocks = b // _BBLK
    n_chunks = t * jblocks // _NW
    idx4 = jnp.transpose(token_ids).reshape(t * jblocks, _BBLK)
    idx4 = idx4.astype(jnp.int32)
    p = _make_gather(n_chunks, jblocks, d)(embed, idx4)
    return p.transpose(2, 4, 0, 1, 3).reshape(b, t, d)
